# Initial kernel scaffold; baseline (speedup 1.0000x reference)
import jax
import jax.numpy as jnp
from jax.experimental import pallas as pl


def kernel(x, edge_index, edge_attr, en1_W1, en1_b1, en1_W2, en1_b2, root1, bias1, en2_W1, en2_b1, en2_W2, en2_b2, root2, bias2, lin_W, lin_b):
    raise NotImplementedError("write your pallas kernel here")


# speedup vs baseline: 1.2185x; 1.2185x over previous
import functools

import jax
import jax.numpy as jnp
from jax.experimental import pallas as pl

N = 50000
E = 800000
IN_CH = 8
HID_CH = 8
EDGE_DIM = 4
NUM_CLASSES = 16

BE = 8000
BN = 5000


def _msg_body(ea_ref, xj_ref, w1_ref, b1_ref, w2_ref, b2_ref, r_ref, s_ref,
              c_ref, o_ref, *, n_valid):
    i = pl.program_id(0)
    ea = ea_ref[...]
    g = jnp.maximum(
        jnp.dot(ea, w1_ref[...], preferred_element_type=jnp.float32)
        + b1_ref[...], 0.0)
    h = jnp.dot(g, w2_ref[...], preferred_element_type=jnp.float32) + b2_ref[...]
    xr = jnp.dot(xj_ref[...], r_ref[...], preferred_element_type=jnp.float32)
    msg = jnp.dot(h * xr, s_ref[...], preferred_element_type=jnp.float32) + c_ref[...]
    blk = ea.shape[0]
    row = i * blk + jax.lax.broadcasted_iota(jnp.int32, (blk, 16), 0)
    o_ref[...] = jnp.where(row < n_valid, msg, 0.0)


def _edge_messages(eaP, xj, W1p, b1p, W2, b2p, n_valid):
    Ep = eaP.shape[0]
    be = BE if Ep % BE == 0 else 8192
    grid = Ep // be
    f32 = jnp.float32
    lane = jnp.arange(16)
    R = (jnp.arange(64)[None, :] // 8 == jnp.arange(16)[:, None]).astype(f32)
    S = ((jnp.arange(64)[:, None] % 8 == lane[None, :])
         & (lane[None, :] < 8)).astype(f32)
    c = (lane == 8).astype(f32)[None, :]
    body = functools.partial(_msg_body, n_valid=n_valid)
    return pl.pallas_call(
        body,
        grid=(grid,),
        in_specs=[
            pl.BlockSpec((be, 8), lambda i: (i, 0)),
            pl.BlockSpec((be, 16), lambda i: (i, 0)),
            pl.BlockSpec((8, 64), lambda i: (0, 0)),
            pl.BlockSpec((1, 64), lambda i: (0, 0)),
            pl.BlockSpec((64, 64), lambda i: (0, 0)),
            pl.BlockSpec((1, 64), lambda i: (0, 0)),
            pl.BlockSpec((16, 64), lambda i: (0, 0)),
            pl.BlockSpec((64, 16), lambda i: (0, 0)),
            pl.BlockSpec((1, 16), lambda i: (0, 0)),
        ],
        out_specs=pl.BlockSpec((be, 16), lambda i: (i, 0)),
        out_shape=jax.ShapeDtypeStruct((Ep, 16), f32),
    )(eaP, xj, W1p, b1p, W2, b2p, R, S, c)


def _agg_body(p_ref, x_ref, root_ref, bias_ref, k_ref, m_ref, lw_ref, lb_ref,
              o_ref, *, final):
    s = p_ref[0] + p_ref[1]
    cnt = jnp.dot(s, k_ref[...], preferred_element_type=jnp.float32)
    recip = 1.0 / jnp.maximum(cnt, 1.0)
    h = jnp.maximum(
        s * recip * m_ref[...]
        + jnp.dot(x_ref[...], root_ref[...], preferred_element_type=jnp.float32)
        + bias_ref[...], 0.0)
    if final:
        o_ref[...] = jnp.dot(h, lw_ref[...],
                             preferred_element_type=jnp.float32) + lb_ref[...]
    else:
        o_ref[...] = h


def _aggregate(p, x16, rootP, biasP, linWP, linb, final):
    f32 = jnp.float32
    lane = jnp.arange(16)
    K = (lane[:, None] == 8).astype(f32) * jnp.ones((16, 16), f32)
    M = (lane < 8).astype(f32)[None, :]
    grid = N // BN
    body = functools.partial(_agg_body, final=final)
    return pl.pallas_call(
        body,
        grid=(grid,),
        in_specs=[
            pl.BlockSpec((2, BN, 16), lambda i: (0, i, 0)),
            pl.BlockSpec((BN, 16), lambda i: (i, 0)),
            pl.BlockSpec((16, 16), lambda i: (0, 0)),
            pl.BlockSpec((1, 16), lambda i: (0, 0)),
            pl.BlockSpec((16, 16), lambda i: (0, 0)),
            pl.BlockSpec((1, 16), lambda i: (0, 0)),
            pl.BlockSpec((16, 16), lambda i: (0, 0)),
            pl.BlockSpec((1, 16), lambda i: (0, 0)),
        ],
        out_specs=pl.BlockSpec((BN, 16), lambda i: (i, 0)),
        out_shape=jax.ShapeDtypeStruct((N, 16), f32),
    )(p, x16, rootP, biasP, K, M, linWP, linb)


def kernel(x, edge_index, edge_attr,
           en1_W1, en1_b1, en1_W2, en1_b2, root1, bias1,
           en2_W1, en2_b1, en2_W2, en2_b2, root2, bias2,
           lin_W, lin_b):
    f32 = jnp.float32
    src = edge_index[0]
    dst = edge_index[1]

    x16 = jnp.pad(x, ((0, 0), (0, 16 - IN_CH)))
    eaP = jnp.pad(edge_attr, ((0, 0), (0, 8 - EDGE_DIM)))
    W1p_1 = jnp.pad(en1_W1, ((0, 8 - EDGE_DIM), (0, 0)))
    W1p_2 = jnp.pad(en2_W1, ((0, 8 - EDGE_DIM), (0, 0)))
    root1P = jnp.pad(root1, ((0, 8), (0, 8)))
    root2P = jnp.pad(root2, ((0, 8), (0, 8)))
    bias1P = jnp.pad(bias1, (0, 8))[None, :]
    bias2P = jnp.pad(bias2, (0, 8))[None, :]
    linWP = jnp.pad(lin_W, ((0, 8), (0, 0)))
    linb = lin_b[None, :]
    zero16 = jnp.zeros((16,), f32)[None, :]

    def layer(table16, W1p, b1, W2, b2, rootP, biasP, final):
        xj = jnp.take(table16, src, axis=0)
        msg = _edge_messages(eaP, xj, W1p, b1[None, :], W2, b2[None, :], E)
        ssum = jax.ops.segment_sum(msg, dst, num_segments=N)
        p = jnp.stack([ssum, jnp.zeros_like(ssum)])
        return _aggregate(p, table16, rootP, biasP,
                          linWP if final else jnp.zeros((16, 16), f32),
                          linb if final else zero16, final)

    h1 = layer(x16, W1p_1, en1_b1, en1_W2, en1_b2, root1P, bias1P, False)
    out = layer(h1, W1p_2, en2_b1, en2_W2, en2_b2, root2P, bias2P, True)
    return out


# speedup vs baseline: 3.3296x; 2.7326x over previous
import functools

import jax
import jax.numpy as jnp
from jax import lax
from jax.experimental import pallas as pl
from jax.experimental.pallas import tpu as pltpu
from jax.experimental.pallas import tpu_sc as plsc

N = 50000
E = 800000
IN_CH = 8
HID_CH = 8
EDGE_DIM = 4
NUM_CLASSES = 16

BE = 8192
BN = 6256

NW = 32
STEPS = 196
GROUP = 14
NG = STEPS // GROUP
EPAD = NW * STEPS * 128
NP = 50048
ROWS_PER_TILE = NP // 16


def _sc_mesh():
    return plsc.VectorSubcoreMesh(core_axis_name="c", subcore_axis_name="s",
                                  num_cores=2, num_subcores=16)


def _sc_gather(table16, idx3):
    @functools.partial(
        pl.kernel, mesh=_sc_mesh(),
        out_type=jax.ShapeDtypeStruct((EPAD, 16), jnp.float32),
        compiler_params=pltpu.CompilerParams(use_tc_tiling_on_sc=False),
        scratch_types=[
            pltpu.VMEM((STEPS, 128), jnp.int32),
            pltpu.VMEM((GROUP * 128, 16), jnp.float32),
            pltpu.VMEM_SHARED((NP, 16), jnp.float32),
            pltpu.SemaphoreType.DMA,
        ])
    def k(table_hbm, idx_hbm, out_hbm, idx_v, stage_v, tbl_sh, sem):
        cid = lax.axis_index("c")
        sid = lax.axis_index("s")
        wid = sid * 2 + cid
        r0 = sid * ROWS_PER_TILE
        pltpu.sync_copy(table_hbm.at[pl.ds(r0, ROWS_PER_TILE)],
                        tbl_sh.at[pl.ds(r0, ROWS_PER_TILE)])
        pltpu.sync_copy(idx_hbm.at[wid], idx_v)
        plsc.subcore_barrier()

        @pl.loop(0, NG)
        def _group(gi):
            descs = [
                pltpu.async_copy(tbl_sh.at[idx_v.at[gi * GROUP + j]],
                                 stage_v.at[pl.ds(j * 128, 128)], sem)
                for j in range(GROUP)
            ]
            for d in descs:
                d.wait()
            base = wid * (STEPS * 128) + gi * (GROUP * 128)
            pltpu.sync_copy(stage_v, out_hbm.at[pl.ds(base, GROUP * 128)])

    return k(table16, idx3)


def _sc_scatter(msg, idx3, zeros):
    @functools.partial(
        pl.kernel, mesh=_sc_mesh(),
        out_type=jax.ShapeDtypeStruct((2, NP, 16), jnp.float32),
        compiler_params=pltpu.CompilerParams(use_tc_tiling_on_sc=False),
        scratch_types=[
            pltpu.VMEM((STEPS, 128), jnp.int32),
            pltpu.VMEM((GROUP * 128, 16), jnp.float32),
            pltpu.VMEM_SHARED((NP, 16), jnp.float32),
        ])
    def k(msg_hbm, idx_hbm, zeros_hbm, out_hbm, idx_v, stage_v, acc_sh):
        cid = lax.axis_index("c")
        sid = lax.axis_index("s")
        wid = sid * 2 + cid
        r0 = sid * ROWS_PER_TILE
        pltpu.sync_copy(zeros_hbm.at[pl.ds(r0, ROWS_PER_TILE)],
                        acc_sh.at[pl.ds(r0, ROWS_PER_TILE)])
        pltpu.sync_copy(idx_hbm.at[wid], idx_v)
        plsc.subcore_barrier()

        @pl.loop(0, NG)
        def _group(gi):
            base = wid * (STEPS * 128) + gi * (GROUP * 128)
            pltpu.sync_copy(msg_hbm.at[pl.ds(base, GROUP * 128)], stage_v)
            for j in range(GROUP):
                pltpu.sync_copy(stage_v.at[pl.ds(j * 128, 128)],
                                acc_sh.at[idx_v.at[gi * GROUP + j]], add=True)

        plsc.subcore_barrier()
        pltpu.sync_copy(acc_sh.at[pl.ds(r0, ROWS_PER_TILE)],
                        out_hbm.at[cid, pl.ds(r0, ROWS_PER_TILE)])

    return k(msg, idx3, zeros)


def _msg_body(ea_ref, xj_ref, w1_ref, b1_ref, w2_ref, b2_ref, r_ref, s_ref,
              c_ref, o_ref, *, n_valid):
    i = pl.program_id(0)
    ea = ea_ref[...]
    g = jnp.maximum(
        jnp.dot(ea, w1_ref[...], preferred_element_type=jnp.float32)
        + b1_ref[...], 0.0)
    h = jnp.dot(g, w2_ref[...], preferred_element_type=jnp.float32) + b2_ref[...]
    xr = jnp.dot(xj_ref[...], r_ref[...], preferred_element_type=jnp.float32)
    msg = jnp.dot(h * xr, s_ref[...], preferred_element_type=jnp.float32) + c_ref[...]
    blk = ea.shape[0]
    row = i * blk + jax.lax.broadcasted_iota(jnp.int32, (blk, 16), 0)
    o_ref[...] = jnp.where(row < n_valid, msg, 0.0)


def _edge_messages(eaP, xj, W1p, b1p, W2, b2p, n_valid):
    Ep = eaP.shape[0]
    be = BE if Ep % BE == 0 else 8192
    grid = Ep // be
    f32 = jnp.float32
    lane = jnp.arange(16)
    R = (jnp.arange(64)[None, :] // 8 == jnp.arange(16)[:, None]).astype(f32)
    S = ((jnp.arange(64)[:, None] % 8 == lane[None, :])
         & (lane[None, :] < 8)).astype(f32)
    c = (lane == 8).astype(f32)[None, :]
    body = functools.partial(_msg_body, n_valid=n_valid)
    return pl.pallas_call(
        body,
        grid=(grid,),
        in_specs=[
            pl.BlockSpec((be, 8), lambda i: (i, 0)),
            pl.BlockSpec((be, 16), lambda i: (i, 0)),
            pl.BlockSpec((8, 64), lambda i: (0, 0)),
            pl.BlockSpec((1, 64), lambda i: (0, 0)),
            pl.BlockSpec((64, 64), lambda i: (0, 0)),
            pl.BlockSpec((1, 64), lambda i: (0, 0)),
            pl.BlockSpec((16, 64), lambda i: (0, 0)),
            pl.BlockSpec((64, 16), lambda i: (0, 0)),
            pl.BlockSpec((1, 16), lambda i: (0, 0)),
        ],
        out_specs=pl.BlockSpec((be, 16), lambda i: (i, 0)),
        out_shape=jax.ShapeDtypeStruct((Ep, 16), f32),
    )(eaP, xj, W1p, b1p, W2, b2p, R, S, c)


def _agg_body(p_ref, x_ref, root_ref, bias_ref, k_ref, m_ref, lw_ref, lb_ref,
              o_ref, *, final):
    s = p_ref[0] + p_ref[1]
    cnt = jnp.dot(s, k_ref[...], preferred_element_type=jnp.float32)
    recip = 1.0 / jnp.maximum(cnt, 1.0)
    h = jnp.maximum(
        s * recip * m_ref[...]
        + jnp.dot(x_ref[...], root_ref[...], preferred_element_type=jnp.float32)
        + bias_ref[...], 0.0)
    if final:
        o_ref[...] = jnp.dot(h, lw_ref[...],
                             preferred_element_type=jnp.float32) + lb_ref[...]
    else:
        o_ref[...] = h


def _aggregate(p, x16, rootP, biasP, linWP, linb, final):
    f32 = jnp.float32
    lane = jnp.arange(16)
    K = (lane[:, None] == 8).astype(f32) * jnp.ones((16, 16), f32)
    M = (lane < 8).astype(f32)[None, :]
    grid = NP // BN
    body = functools.partial(_agg_body, final=final)
    return pl.pallas_call(
        body,
        grid=(grid,),
        in_specs=[
            pl.BlockSpec((2, BN, 16), lambda i: (0, i, 0)),
            pl.BlockSpec((BN, 16), lambda i: (i, 0)),
            pl.BlockSpec((16, 16), lambda i: (0, 0)),
            pl.BlockSpec((1, 16), lambda i: (0, 0)),
            pl.BlockSpec((16, 16), lambda i: (0, 0)),
            pl.BlockSpec((1, 16), lambda i: (0, 0)),
            pl.BlockSpec((16, 16), lambda i: (0, 0)),
            pl.BlockSpec((1, 16), lambda i: (0, 0)),
        ],
        out_specs=pl.BlockSpec((BN, 16), lambda i: (i, 0)),
        out_shape=jax.ShapeDtypeStruct((NP, 16), f32),
    )(p, x16, rootP, biasP, K, M, linWP, linb)


def kernel(x, edge_index, edge_attr,
           en1_W1, en1_b1, en1_W2, en1_b2, root1, bias1,
           en2_W1, en2_b1, en2_W2, en2_b2, root2, bias2,
           lin_W, lin_b):
    f32 = jnp.float32
    src = edge_index[0]
    dst = edge_index[1]

    PAD = EPAD - E
    padidx = (jnp.arange(PAD, dtype=jnp.int32) * 61) % N
    src3 = jnp.concatenate([src, padidx]).reshape(NW, STEPS, 128)
    dst3 = jnp.concatenate([dst, padidx]).reshape(NW, STEPS, 128)
    zeros16 = jnp.zeros((NP, 16), f32)
    x16 = jnp.pad(x, ((0, NP - N), (0, 16 - IN_CH)))
    eaP = jnp.pad(edge_attr, ((0, PAD), (0, 8 - EDGE_DIM)))
    W1p_1 = jnp.pad(en1_W1, ((0, 8 - EDGE_DIM), (0, 0)))
    W1p_2 = jnp.pad(en2_W1, ((0, 8 - EDGE_DIM), (0, 0)))
    root1P = jnp.pad(root1, ((0, 8), (0, 8)))
    root2P = jnp.pad(root2, ((0, 8), (0, 8)))
    bias1P = jnp.pad(bias1, (0, 8))[None, :]
    bias2P = jnp.pad(bias2, (0, 8))[None, :]
    linWP = jnp.pad(lin_W, ((0, 8), (0, 0)))
    linb = lin_b[None, :]
    zero16 = jnp.zeros((16,), f32)[None, :]

    def layer(table16, W1p, b1, W2, b2, rootP, biasP, final):
        xj = _sc_gather(table16, src3)
        msg = _edge_messages(eaP, xj, W1p, b1[None, :], W2, b2[None, :], E)
        p = _sc_scatter(msg, dst3, zeros16)
        return _aggregate(p, table16, rootP, biasP,
                          linWP if final else jnp.zeros((16, 16), f32),
                          linb if final else zero16, final)

    h1 = layer(x16, W1p_1, en1_b1, en1_W2, en1_b2, root1P, bias1P, False)
    out = layer(h1, W1p_2, en2_b1, en2_W2, en2_b2, root2P, bias2P, True)
    return out[:N]


# speedup vs baseline: 3.9810x; 1.1956x over previous
import functools

import jax
import jax.numpy as jnp
from jax import lax
from jax.experimental import pallas as pl
from jax.experimental.pallas import tpu as pltpu
from jax.experimental.pallas import tpu_sc as plsc

N = 50000
E = 800000
IN_CH = 8
HID_CH = 8
EDGE_DIM = 4
NUM_CLASSES = 16

NW = 32
STEPS = 196
GROUP = 14
NG = STEPS // GROUP
EPAD = NW * STEPS * 128
NP = 50176
ROWS_PER_TILE = NP // 16
TR8 = NP // 8 // 16
CE = GROUP * 16
CN = TR8 // 7
BE = GROUP * 128

f32 = jnp.float32


def _sc_mesh():
    return plsc.VectorSubcoreMesh(core_axis_name="c", subcore_axis_name="s",
                                  num_cores=2, num_subcores=16)


def _sc_gather(t128, idx3):
    @functools.partial(
        pl.kernel, mesh=_sc_mesh(),
        out_type=jax.ShapeDtypeStruct((EPAD // 8, 128), f32),
        compiler_params=pltpu.CompilerParams(use_tc_tiling_on_sc=False),
        scratch_types=[
            pltpu.VMEM((GROUP, 128), jnp.int32),
            pltpu.VMEM((GROUP * 128, 16), f32),
            pltpu.VMEM((CE, 128), f32),
            pltpu.VMEM_SHARED((NP, 16), f32),
            pltpu.SemaphoreType.DMA,
        ])
    def k(tbl_hbm, idx_hbm, out_hbm, idx_v, stage_v, pack_v, tbl_sh, sem):
        cid = lax.axis_index("c")
        sid = lax.axis_index("s")
        wid = sid * 2 + cid
        r8 = sid * TR8

        @pl.loop(0, 7)
        def _tstage(ti):
            woff = r8 + ti * CN
            pltpu.sync_copy(tbl_hbm.at[pl.ds(woff, CN)],
                            pack_v.at[pl.ds(0, CN)])

            @pl.loop(0, CN, unroll=8)
            def _trepack(q):
                for c in range(8):
                    stage_v[c * CN + q, :] = pack_v[q, pl.ds(c * 16, 16)]

            pltpu.sync_copy(stage_v.at[pl.ds(0, CN * 8)],
                            tbl_sh.at[pl.ds(woff * 8, CN * 8)])

        plsc.subcore_barrier()

        @pl.loop(0, NG)
        def _group(gi):
            pltpu.sync_copy(idx_hbm.at[wid, pl.ds(gi * GROUP, GROUP)], idx_v)
            descs = [
                pltpu.async_copy(tbl_sh.at[idx_v.at[j]],
                                 stage_v.at[pl.ds(j * 128, 128)], sem)
                for j in range(GROUP)
            ]
            for d in descs:
                d.wait()

            @pl.loop(0, CE, unroll=8)
            def _repack(q):
                for c in range(8):
                    pack_v[q, pl.ds(c * 16, 16)] = stage_v[c * CE + q, :]

            base8 = wid * (STEPS * 16) + gi * CE
            pltpu.sync_copy(pack_v, out_hbm.at[pl.ds(base8, CE)])

    return k(t128, idx3)


def _sc_scatter(m128, idx3):
    @functools.partial(
        pl.kernel, mesh=_sc_mesh(),
        out_type=jax.ShapeDtypeStruct((2, NP // 8, 128), f32),
        compiler_params=pltpu.CompilerParams(use_tc_tiling_on_sc=False),
        scratch_types=[
            pltpu.VMEM((GROUP, 128), jnp.int32),
            pltpu.VMEM((GROUP * 128, 16), f32),
            pltpu.VMEM((CE, 128), f32),
            pltpu.VMEM_SHARED((NP, 16), f32),
        ])
    def k(msg_hbm, idx_hbm, out_hbm, idx_v, stage_v, pack_v, acc_sh):
        cid = lax.axis_index("c")
        sid = lax.axis_index("s")
        wid = sid * 2 + cid
        r8 = sid * TR8

        @pl.loop(0, GROUP * 128, unroll=8)
        def _zero(q):
            stage_v[q, :] = jnp.zeros((16,), f32)

        pltpu.sync_copy(stage_v.at[pl.ds(0, TR8 * 4)],
                        acc_sh.at[pl.ds(r8 * 8, TR8 * 4)])
        pltpu.sync_copy(stage_v.at[pl.ds(0, TR8 * 4)],
                        acc_sh.at[pl.ds(r8 * 8 + TR8 * 4, TR8 * 4)])
        plsc.subcore_barrier()

        @pl.loop(0, NG)
        def _group(gi):
            pltpu.sync_copy(idx_hbm.at[wid, pl.ds(gi * GROUP, GROUP)], idx_v)
            base8 = wid * (STEPS * 16) + gi * CE
            pltpu.sync_copy(msg_hbm.at[pl.ds(base8, CE)], pack_v)

            @pl.loop(0, CE, unroll=8)
            def _repack(q):
                for c in range(8):
                    stage_v[c * CE + q, :] = pack_v[q, pl.ds(c * 16, 16)]

            for j in range(GROUP):
                pltpu.sync_copy(stage_v.at[pl.ds(j * 128, 128)],
                                acc_sh.at[idx_v.at[j]], add=True)

        plsc.subcore_barrier()

        @pl.loop(0, 7)
        def _out(ti):
            woff = r8 + ti * CN
            pltpu.sync_copy(acc_sh.at[pl.ds(woff * 8, CN * 8)],
                            stage_v.at[pl.ds(0, CN * 8)])

            @pl.loop(0, CN, unroll=8)
            def _orepack(q):
                for c in range(8):
                    pack_v[q, pl.ds(c * 16, 16)] = stage_v[c * CN + q, :]

            pltpu.sync_copy(pack_v.at[pl.ds(0, CN)],
                            out_hbm.at[cid, pl.ds(woff, CN)])

    return k(m128, idx3)


def _msg_body(ea_ref, xj_ref, w1_ref, b1_ref, w2_ref, b2_ref, r_ref, s_ref,
              c_ref, o_ref, *, n_valid):
    i = pl.program_id(0)
    ea = ea_ref[...]
    xj128 = xj_ref[...]
    xj = jnp.concatenate([xj128[:, c * 16:(c + 1) * 16] for c in range(8)],
                         axis=0)
    g = jnp.maximum(
        jnp.dot(ea, w1_ref[...], preferred_element_type=f32) + b1_ref[...],
        0.0)
    h = jnp.dot(g, w2_ref[...], preferred_element_type=f32) + b2_ref[...]
    xr = jnp.dot(xj, r_ref[...], preferred_element_type=f32)
    msg = jnp.dot(h * xr, s_ref[...], preferred_element_type=f32) + c_ref[...]
    row = i * BE + jax.lax.broadcasted_iota(jnp.int32, (BE, 16), 0)
    msg = jnp.where(row < n_valid, msg, 0.0)
    o_ref[...] = jnp.concatenate(
        [msg[c * CE:(c + 1) * CE, :] for c in range(8)], axis=1)


def _edge_messages(eaP, xj128, W1p, b1p, W2, b2p, n_valid):
    grid = EPAD // BE
    lane = jnp.arange(16)
    R = (jnp.arange(64)[None, :] // 8 == jnp.arange(16)[:, None]).astype(f32)
    S = ((jnp.arange(64)[:, None] % 8 == lane[None, :])
         & (lane[None, :] < 8)).astype(f32)
    c = (lane == 8).astype(f32)[None, :]
    body = functools.partial(_msg_body, n_valid=n_valid)
    return pl.pallas_call(
        body,
        grid=(grid,),
        in_specs=[
            pl.BlockSpec((BE, 8), lambda i: (i, 0)),
            pl.BlockSpec((CE, 128), lambda i: (i, 0)),
            pl.BlockSpec((8, 64), lambda i: (0, 0)),
            pl.BlockSpec((1, 64), lambda i: (0, 0)),
            pl.BlockSpec((64, 64), lambda i: (0, 0)),
            pl.BlockSpec((1, 64), lambda i: (0, 0)),
            pl.BlockSpec((16, 64), lambda i: (0, 0)),
            pl.BlockSpec((64, 16), lambda i: (0, 0)),
            pl.BlockSpec((1, 16), lambda i: (0, 0)),
        ],
        out_specs=pl.BlockSpec((CE, 128), lambda i: (i, 0)),
        out_shape=jax.ShapeDtypeStruct((EPAD // 8, 128), f32),
    )(eaP, xj128, W1p, b1p, W2, b2p, R, S, c)


def _agg_body(p_ref, x_ref, root_ref, bias_ref, k_ref, m_ref, lw_ref, lb_ref,
              o_ref, *, final):
    def unpack(w):
        return jnp.concatenate([w[:, c * 16:(c + 1) * 16] for c in range(8)],
                               axis=0)

    s = unpack(p_ref[0]) + unpack(p_ref[1])
    cnt = jnp.dot(s, k_ref[...], preferred_element_type=f32)
    recip = 1.0 / jnp.maximum(cnt, 1.0)
    x16 = unpack(x_ref[...])
    h = jnp.maximum(
        s * recip * m_ref[...]
        + jnp.dot(x16, root_ref[...], preferred_element_type=f32)
        + bias_ref[...], 0.0)
    if final:
        o_ref[...] = jnp.dot(h, lw_ref[...],
                             preferred_element_type=f32) + lb_ref[...]
    else:
        o_ref[...] = jnp.concatenate(
            [h[c * CN:(c + 1) * CN, :] for c in range(8)], axis=1)


def _aggregate(p, x128, rootP, biasP, linWP, linb, final):
    lane = jnp.arange(16)
    K = (lane[:, None] == 8).astype(f32) * jnp.ones((16, 16), f32)
    M = (lane < 8).astype(f32)[None, :]
    grid = NP // 8 // CN
    body = functools.partial(_agg_body, final=final)
    if final:
        out_spec = pl.BlockSpec((CN * 8, 16), lambda i: (i, 0))
        out_shape = jax.ShapeDtypeStruct((NP, 16), f32)
    else:
        out_spec = pl.BlockSpec((CN, 128), lambda i: (i, 0))
        out_shape = jax.ShapeDtypeStruct((NP // 8, 128), f32)
    return pl.pallas_call(
        body,
        grid=(grid,),
        in_specs=[
            pl.BlockSpec((2, CN, 128), lambda i: (0, i, 0)),
            pl.BlockSpec((CN, 128), lambda i: (i, 0)),
            pl.BlockSpec((16, 16), lambda i: (0, 0)),
            pl.BlockSpec((1, 16), lambda i: (0, 0)),
            pl.BlockSpec((16, 16), lambda i: (0, 0)),
            pl.BlockSpec((1, 16), lambda i: (0, 0)),
            pl.BlockSpec((16, 16), lambda i: (0, 0)),
            pl.BlockSpec((1, 16), lambda i: (0, 0)),
        ],
        out_specs=out_spec,
        out_shape=out_shape,
    )(p, x128, rootP, biasP, K, M, linWP, linb)


def _pack_nodes(a16):
    return (a16.reshape(NP // 8 // CN, 8, CN, 16)
            .transpose(0, 2, 1, 3)
            .reshape(NP // 8, 128))


def kernel(x, edge_index, edge_attr,
           en1_W1, en1_b1, en1_W2, en1_b2, root1, bias1,
           en2_W1, en2_b1, en2_W2, en2_b2, root2, bias2,
           lin_W, lin_b):
    src = edge_index[0]
    dst = edge_index[1]

    PAD = EPAD - E
    padidx = (jnp.arange(PAD, dtype=jnp.int32) * 61) % N
    src3 = jnp.concatenate([src, padidx]).reshape(NW, STEPS, 128)
    dst3 = jnp.concatenate([dst, padidx]).reshape(NW, STEPS, 128)
    x128 = _pack_nodes(jnp.pad(x, ((0, NP - N), (0, 16 - IN_CH))))
    eaP = jnp.pad(edge_attr, ((0, PAD), (0, 8 - EDGE_DIM)))
    W1p_1 = jnp.pad(en1_W1, ((0, 8 - EDGE_DIM), (0, 0)))
    W1p_2 = jnp.pad(en2_W1, ((0, 8 - EDGE_DIM), (0, 0)))
    root1P = jnp.pad(root1, ((0, 8), (0, 8)))
    root2P = jnp.pad(root2, ((0, 8), (0, 8)))
    bias1P = jnp.pad(bias1, (0, 8))[None, :]
    bias2P = jnp.pad(bias2, (0, 8))[None, :]
    linWP = jnp.pad(lin_W, ((0, 8), (0, 0)))
    linb = lin_b[None, :]
    zero16 = jnp.zeros((16,), f32)[None, :]

    def layer(t128, W1p, b1, W2, b2, rootP, biasP, final):
        xj128 = _sc_gather(t128, src3)
        msg = _edge_messages(eaP, xj128, W1p, b1[None, :], W2, b2[None, :], E)
        p = _sc_scatter(msg, dst3)
        return _aggregate(p, t128, rootP, biasP,
                          linWP if final else jnp.zeros((16, 16), f32),
                          linb if final else zero16, final)

    h1 = layer(x128, W1p_1, en1_b1, en1_W2, en1_b2, root1P, bias1P, False)
    out = layer(h1, W1p_2, en2_b1, en2_W2, en2_b2, root2P, bias2P, True)
    return out[:N]


# speedup vs baseline: 5.9449x; 1.4933x over previous
import functools

import jax
import jax.numpy as jnp
from jax import lax
from jax.experimental import pallas as pl
from jax.experimental.pallas import tpu as pltpu
from jax.experimental.pallas import tpu_sc as plsc

N = 50000
E = 800000
IN_CH = 8
HID_CH = 8
EDGE_DIM = 4
NUM_CLASSES = 16

NW = 32
STEPS = 196
GROUP = 14
NG = STEPS // GROUP
EPAD = NW * STEPS * 128
NP = 50176
ROWS_PER_TILE = NP // 16
TR8 = NP // 8 // 16
CE = GROUP * 16
CN = TR8 // 7
BE = GROUP * 128

f32 = jnp.float32


def _sc_mesh():
    return plsc.VectorSubcoreMesh(core_axis_name="c", subcore_axis_name="s",
                                  num_cores=2, num_subcores=16)


def _sc_gather(t128, idx3):
    @functools.partial(
        pl.kernel, mesh=_sc_mesh(),
        out_type=jax.ShapeDtypeStruct((EPAD // 8, 128), f32),
        compiler_params=pltpu.CompilerParams(use_tc_tiling_on_sc=False),
        scratch_types=[
            pltpu.VMEM((GROUP, 128), jnp.int32),
            pltpu.VMEM((GROUP * 128, 16), f32),
            pltpu.VMEM((CE, 128), f32),
            pltpu.VMEM_SHARED((NP, 16), f32),
            pltpu.SemaphoreType.DMA,
        ])
    def k(tbl_hbm, idx_hbm, out_hbm, idx_v, stage_v, pack_v, tbl_sh, sem):
        cid = lax.axis_index("c")
        sid = lax.axis_index("s")
        wid = sid * 2 + cid
        r8 = sid * TR8

        @pl.loop(0, 7)
        def _tstage(ti):
            woff = r8 + ti * CN
            pltpu.sync_copy(tbl_hbm.at[pl.ds(woff, CN)],
                            pack_v.at[pl.ds(0, CN)])

            @pl.loop(0, CN, unroll=8)
            def _trepack(q):
                for c in range(8):
                    stage_v[c * CN + q, :] = pack_v[q, pl.ds(c * 16, 16)]

            pltpu.sync_copy(stage_v.at[pl.ds(0, CN * 8)],
                            tbl_sh.at[pl.ds(woff * 8, CN * 8)])

        plsc.subcore_barrier()

        @pl.loop(0, NG)
        def _group(gi):
            pltpu.sync_copy(
                idx_hbm.at[pl.ds(wid * STEPS + gi * GROUP, GROUP)], idx_v)
            descs = [
                pltpu.async_copy(tbl_sh.at[idx_v.at[j]],
                                 stage_v.at[pl.ds(j * 128, 128)], sem)
                for j in range(GROUP)
            ]
            for d in descs:
                d.wait()

            @pl.loop(0, CE, unroll=8)
            def _repack(q):
                for c in range(8):
                    pack_v[q, pl.ds(c * 16, 16)] = stage_v[c * CE + q, :]

            base8 = wid * (STEPS * 16) + gi * CE
            pltpu.sync_copy(pack_v, out_hbm.at[pl.ds(base8, CE)])

    return k(t128, idx3)


def _sc_scatter(m128, idx3):
    @functools.partial(
        pl.kernel, mesh=_sc_mesh(),
        out_type=jax.ShapeDtypeStruct((2, NP // 8, 128), f32),
        compiler_params=pltpu.CompilerParams(use_tc_tiling_on_sc=False),
        scratch_types=[
            pltpu.VMEM((GROUP, 128), jnp.int32),
            pltpu.VMEM((GROUP * 128, 16), f32),
            pltpu.VMEM((CE, 128), f32),
            pltpu.VMEM_SHARED((NP, 16), f32),
        ])
    def k(msg_hbm, idx_hbm, out_hbm, idx_v, stage_v, pack_v, acc_sh):
        cid = lax.axis_index("c")
        sid = lax.axis_index("s")
        wid = sid * 2 + cid
        r8 = sid * TR8

        @pl.loop(0, GROUP * 128, unroll=8)
        def _zero(q):
            stage_v[q, :] = jnp.zeros((16,), f32)

        pltpu.sync_copy(stage_v.at[pl.ds(0, TR8 * 4)],
                        acc_sh.at[pl.ds(r8 * 8, TR8 * 4)])
        pltpu.sync_copy(stage_v.at[pl.ds(0, TR8 * 4)],
                        acc_sh.at[pl.ds(r8 * 8 + TR8 * 4, TR8 * 4)])
        plsc.subcore_barrier()

        @pl.loop(0, NG)
        def _group(gi):
            pltpu.sync_copy(
                idx_hbm.at[pl.ds(wid * STEPS + gi * GROUP, GROUP)], idx_v)
            base8 = wid * (STEPS * 16) + gi * CE
            pltpu.sync_copy(msg_hbm.at[pl.ds(base8, CE)], pack_v)

            @pl.loop(0, CE, unroll=8)
            def _repack(q):
                for c in range(8):
                    stage_v[c * CE + q, :] = pack_v[q, pl.ds(c * 16, 16)]

            for j in range(GROUP):
                pltpu.sync_copy(stage_v.at[pl.ds(j * 128, 128)],
                                acc_sh.at[idx_v.at[j]], add=True)

        plsc.subcore_barrier()

        @pl.loop(0, 7)
        def _out(ti):
            woff = r8 + ti * CN
            pltpu.sync_copy(acc_sh.at[pl.ds(woff * 8, CN * 8)],
                            stage_v.at[pl.ds(0, CN * 8)])

            @pl.loop(0, CN, unroll=8)
            def _orepack(q):
                for c in range(8):
                    pack_v[q, pl.ds(c * 16, 16)] = stage_v[c * CN + q, :]

            pltpu.sync_copy(pack_v.at[pl.ds(0, CN)],
                            out_hbm.at[cid, pl.ds(woff, CN)])

    return k(m128, idx3)


def _msg_body(ea_ref, xj_ref, w1_ref, b1_ref, w2_ref, b2_ref, r_ref, s_ref,
              c_ref, o_ref, *, n_valid):
    i = pl.program_id(0)
    ea128 = ea_ref[...]
    ea = jnp.concatenate([ea128[:, c * 4:(c + 1) * 4] for c in range(32)],
                         axis=0)
    xj128 = xj_ref[...]
    xj = jnp.concatenate([xj128[:, c * 16:(c + 1) * 16] for c in range(8)],
                         axis=0)
    g = jnp.maximum(
        jnp.dot(ea, w1_ref[...], preferred_element_type=f32) + b1_ref[...],
        0.0)
    h = jnp.dot(g, w2_ref[...], preferred_element_type=f32) + b2_ref[...]
    xr = jnp.dot(xj, r_ref[...], preferred_element_type=f32)
    msg = jnp.dot(h * xr, s_ref[...], preferred_element_type=f32) + c_ref[...]
    row = i * BE + jax.lax.broadcasted_iota(jnp.int32, (BE, 16), 0)
    msg = jnp.where(row < n_valid, msg, 0.0)
    o_ref[...] = jnp.concatenate(
        [msg[c * CE:(c + 1) * CE, :] for c in range(8)], axis=1)


def _edge_messages(ea128, xj128, W1p, b1p, W2, b2p, n_valid):
    grid = EPAD // BE
    lane = jnp.arange(16)
    R = (jnp.arange(64)[None, :] // 8 == jnp.arange(16)[:, None]).astype(f32)
    S = ((jnp.arange(64)[:, None] % 8 == lane[None, :])
         & (lane[None, :] < 8)).astype(f32)
    c = (lane == 8).astype(f32)[None, :]
    body = functools.partial(_msg_body, n_valid=n_valid)
    return pl.pallas_call(
        body,
        grid=(grid,),
        in_specs=[
            pl.BlockSpec((BE // 32, 128), lambda i: (i, 0)),
            pl.BlockSpec((CE, 128), lambda i: (i, 0)),
            pl.BlockSpec((4, 64), lambda i: (0, 0)),
            pl.BlockSpec((1, 64), lambda i: (0, 0)),
            pl.BlockSpec((64, 64), lambda i: (0, 0)),
            pl.BlockSpec((1, 64), lambda i: (0, 0)),
            pl.BlockSpec((16, 64), lambda i: (0, 0)),
            pl.BlockSpec((64, 16), lambda i: (0, 0)),
            pl.BlockSpec((1, 16), lambda i: (0, 0)),
        ],
        out_specs=pl.BlockSpec((CE, 128), lambda i: (i, 0)),
        out_shape=jax.ShapeDtypeStruct((EPAD // 8, 128), f32),
    )(ea128, xj128, W1p, b1p, W2, b2p, R, S, c)


def _agg_body(p_ref, x_ref, root_ref, bias_ref, k_ref, m_ref, lw_ref, lb_ref,
              o_ref, *, final):
    def unpack(w):
        return jnp.concatenate([w[:, c * 16:(c + 1) * 16] for c in range(8)],
                               axis=0)

    s = unpack(p_ref[0]) + unpack(p_ref[1])
    cnt = jnp.dot(s, k_ref[...], preferred_element_type=f32)
    recip = 1.0 / jnp.maximum(cnt, 1.0)
    x16 = unpack(x_ref[...])
    h = jnp.maximum(
        s * recip * m_ref[...]
        + jnp.dot(x16, root_ref[...], preferred_element_type=f32)
        + bias_ref[...], 0.0)
    if final:
        o_ref[...] = jnp.dot(h, lw_ref[...],
                             preferred_element_type=f32) + lb_ref[...]
    else:
        o_ref[...] = jnp.concatenate(
            [h[c * CN:(c + 1) * CN, :] for c in range(8)], axis=1)


def _aggregate(p, x128, rootP, biasP, linWP, linb, final):
    lane = jnp.arange(16)
    K = (lane[:, None] == 8).astype(f32) * jnp.ones((16, 16), f32)
    M = (lane < 8).astype(f32)[None, :]
    grid = NP // 8 // CN
    body = functools.partial(_agg_body, final=final)
    if final:
        out_spec = pl.BlockSpec((CN * 8, 16), lambda i: (i, 0))
        out_shape = jax.ShapeDtypeStruct((NP, 16), f32)
    else:
        out_spec = pl.BlockSpec((CN, 128), lambda i: (i, 0))
        out_shape = jax.ShapeDtypeStruct((NP // 8, 128), f32)
    return pl.pallas_call(
        body,
        grid=(grid,),
        in_specs=[
            pl.BlockSpec((2, CN, 128), lambda i: (0, i, 0)),
            pl.BlockSpec((CN, 128), lambda i: (i, 0)),
            pl.BlockSpec((16, 16), lambda i: (0, 0)),
            pl.BlockSpec((1, 16), lambda i: (0, 0)),
            pl.BlockSpec((16, 16), lambda i: (0, 0)),
            pl.BlockSpec((1, 16), lambda i: (0, 0)),
            pl.BlockSpec((16, 16), lambda i: (0, 0)),
            pl.BlockSpec((1, 16), lambda i: (0, 0)),
        ],
        out_specs=out_spec,
        out_shape=out_shape,
    )(p, x128, rootP, biasP, K, M, linWP, linb)


def _pack_nodes(a16):
    return (a16.reshape(NP // 8 // CN, 8, CN, 16)
            .transpose(0, 2, 1, 3)
            .reshape(NP // 8, 128))


def kernel(x, edge_index, edge_attr,
           en1_W1, en1_b1, en1_W2, en1_b2, root1, bias1,
           en2_W1, en2_b1, en2_W2, en2_b2, root2, bias2,
           lin_W, lin_b):
    src = edge_index[0]
    dst = edge_index[1]

    PAD = EPAD - E
    padidx = (jnp.arange(PAD, dtype=jnp.int32) * 61) % N
    src3 = jnp.concatenate([src, padidx]).reshape(NW * STEPS, 128)
    dst3 = jnp.concatenate([dst, padidx]).reshape(NW * STEPS, 128)
    x128 = _pack_nodes(jnp.pad(x, ((0, NP - N), (0, 16 - IN_CH))))
    ea128 = (jnp.pad(edge_attr, ((0, PAD), (0, 0)))
             .reshape(EPAD // BE, 32, BE // 32, EDGE_DIM)
             .transpose(0, 2, 1, 3)
             .reshape(EPAD // 32, 128))
    W1p_1 = en1_W1
    W1p_2 = en2_W1
    root1P = jnp.pad(root1, ((0, 8), (0, 8)))
    root2P = jnp.pad(root2, ((0, 8), (0, 8)))
    bias1P = jnp.pad(bias1, (0, 8))[None, :]
    bias2P = jnp.pad(bias2, (0, 8))[None, :]
    linWP = jnp.pad(lin_W, ((0, 8), (0, 0)))
    linb = lin_b[None, :]
    zero16 = jnp.zeros((16,), f32)[None, :]

    def layer(t128, W1p, b1, W2, b2, rootP, biasP, final):
        xj128 = _sc_gather(t128, src3)
        msg = _edge_messages(ea128, xj128, W1p, b1[None, :], W2, b2[None, :], E)
        p = _sc_scatter(msg, dst3)
        return _aggregate(p, t128, rootP, biasP,
                          linWP if final else jnp.zeros((16, 16), f32),
                          linb if final else zero16, final)

    h1 = layer(x128, W1p_1, en1_b1, en1_W2, en1_b2, root1P, bias1P, False)
    out = layer(h1, W1p_2, en2_b1, en2_W2, en2_b2, root2P, bias2P, True)
    return out[:N]


# speedup vs baseline: 5.9522x; 1.0012x over previous
import functools

import jax
import jax.numpy as jnp
from jax import lax
from jax.experimental import pallas as pl
from jax.experimental.pallas import tpu as pltpu
from jax.experimental.pallas import tpu_sc as plsc

N = 50000
E = 800000
IN_CH = 8
HID_CH = 8
EDGE_DIM = 4
NUM_CLASSES = 16

NW = 32
STEPS = 196
GROUP = 14
NG = STEPS // GROUP
EPAD = NW * STEPS * 128
NP = 50176
ROWS_PER_TILE = NP // 16
TR8 = NP // 8 // 16
CE = GROUP * 16
CN = TR8 // 7
BE = GROUP * 128

f32 = jnp.float32


def _sc_mesh():
    return plsc.VectorSubcoreMesh(core_axis_name="c", subcore_axis_name="s",
                                  num_cores=2, num_subcores=16)


def _sc_gather(t128, idx3):
    @functools.partial(
        pl.kernel, mesh=_sc_mesh(),
        out_type=jax.ShapeDtypeStruct((EPAD // 8, 128), f32),
        compiler_params=pltpu.CompilerParams(use_tc_tiling_on_sc=False),
        scratch_types=[
            pltpu.VMEM((GROUP, 128), jnp.int32),
            pltpu.VMEM((GROUP * 128, 16), f32),
            pltpu.VMEM((CE, 128), f32),
            pltpu.VMEM_SHARED((NP, 16), f32),
            pltpu.SemaphoreType.DMA,
        ])
    def k(tbl_hbm, idx_hbm, out_hbm, idx_v, stage_v, pack_v, tbl_sh, sem):
        cid = lax.axis_index("c")
        sid = lax.axis_index("s")
        wid = sid * 2 + cid
        r8 = sid * TR8

        @pl.loop(0, 7)
        def _tstage(ti):
            woff = r8 + ti * CN
            pltpu.sync_copy(tbl_hbm.at[pl.ds(woff, CN)],
                            pack_v.at[pl.ds(0, CN)])

            @pl.loop(0, CN, unroll=14)
            def _trepack(q):
                for c in range(8):
                    stage_v[c * CN + q, :] = pack_v[q, pl.ds(c * 16, 16)]

            pltpu.sync_copy(stage_v.at[pl.ds(0, CN * 8)],
                            tbl_sh.at[pl.ds(woff * 8, CN * 8)])

        plsc.subcore_barrier()

        @pl.loop(0, NG)
        def _group(gi):
            pltpu.sync_copy(
                idx_hbm.at[pl.ds(wid * STEPS + gi * GROUP, GROUP)], idx_v)
            descs = [
                pltpu.async_copy(tbl_sh.at[idx_v.at[j]],
                                 stage_v.at[pl.ds(j * 128, 128)], sem)
                for j in range(GROUP)
            ]
            for d in descs:
                d.wait()

            @pl.loop(0, CE, unroll=16)
            def _repack(q):
                for c in range(8):
                    pack_v[q, pl.ds(c * 16, 16)] = stage_v[c * CE + q, :]

            base8 = wid * (STEPS * 16) + gi * CE
            pltpu.sync_copy(pack_v, out_hbm.at[pl.ds(base8, CE)])

    return k(t128, idx3)


def _sc_scatter(m128, idx3):
    @functools.partial(
        pl.kernel, mesh=_sc_mesh(),
        out_type=jax.ShapeDtypeStruct((2, NP // 8, 128), f32),
        compiler_params=pltpu.CompilerParams(use_tc_tiling_on_sc=False),
        scratch_types=[
            pltpu.VMEM((GROUP, 128), jnp.int32),
            pltpu.VMEM((GROUP * 128, 16), f32),
            pltpu.VMEM((CE, 128), f32),
            pltpu.VMEM_SHARED((NP, 16), f32),
        ])
    def k(msg_hbm, idx_hbm, out_hbm, idx_v, stage_v, pack_v, acc_sh):
        cid = lax.axis_index("c")
        sid = lax.axis_index("s")
        wid = sid * 2 + cid
        r8 = sid * TR8

        @pl.loop(0, GROUP * 128, unroll=16)
        def _zero(q):
            stage_v[q, :] = jnp.zeros((16,), f32)

        pltpu.sync_copy(stage_v.at[pl.ds(0, TR8 * 4)],
                        acc_sh.at[pl.ds(r8 * 8, TR8 * 4)])
        pltpu.sync_copy(stage_v.at[pl.ds(0, TR8 * 4)],
                        acc_sh.at[pl.ds(r8 * 8 + TR8 * 4, TR8 * 4)])
        plsc.subcore_barrier()

        @pl.loop(0, NG)
        def _group(gi):
            pltpu.sync_copy(
                idx_hbm.at[pl.ds(wid * STEPS + gi * GROUP, GROUP)], idx_v)
            base8 = wid * (STEPS * 16) + gi * CE
            pltpu.sync_copy(msg_hbm.at[pl.ds(base8, CE)], pack_v)

            @pl.loop(0, CE, unroll=16)
            def _repack(q):
                for c in range(8):
                    stage_v[c * CE + q, :] = pack_v[q, pl.ds(c * 16, 16)]

            for j in range(GROUP):
                pltpu.sync_copy(stage_v.at[pl.ds(j * 128, 128)],
                                acc_sh.at[idx_v.at[j]], add=True)

        plsc.subcore_barrier()

        @pl.loop(0, 7)
        def _out(ti):
            woff = r8 + ti * CN
            pltpu.sync_copy(acc_sh.at[pl.ds(woff * 8, CN * 8)],
                            stage_v.at[pl.ds(0, CN * 8)])

            @pl.loop(0, CN, unroll=14)
            def _orepack(q):
                for c in range(8):
                    pack_v[q, pl.ds(c * 16, 16)] = stage_v[c * CN + q, :]

            pltpu.sync_copy(pack_v.at[pl.ds(0, CN)],
                            out_hbm.at[cid, pl.ds(woff, CN)])

    return k(m128, idx3)


def _msg_body(ea_ref, xj_ref, w1_ref, b1_ref, w2_ref, b2_ref, r_ref, s_ref,
              c_ref, o_ref, *, n_valid):
    i = pl.program_id(0)
    ea128 = ea_ref[...]
    ea = jnp.concatenate([ea128[:, c * 4:(c + 1) * 4] for c in range(32)],
                         axis=0)
    xj128 = xj_ref[...]
    xj = jnp.concatenate([xj128[:, c * 16:(c + 1) * 16] for c in range(8)],
                         axis=0)
    g = jnp.maximum(
        jnp.dot(ea, w1_ref[...], preferred_element_type=f32) + b1_ref[...],
        0.0)
    h = jnp.dot(g, w2_ref[...], preferred_element_type=f32) + b2_ref[...]
    xr = jnp.dot(xj, r_ref[...], preferred_element_type=f32)
    msg = jnp.dot(h * xr, s_ref[...], preferred_element_type=f32) + c_ref[...]
    row = i * BE + jax.lax.broadcasted_iota(jnp.int32, (BE, 16), 0)
    msg = jnp.where(row < n_valid, msg, 0.0)
    o_ref[...] = jnp.concatenate(
        [msg[c * CE:(c + 1) * CE, :] for c in range(8)], axis=1)


def _edge_messages(ea128, xj128, W1p, b1p, W2, b2p, n_valid):
    grid = EPAD // BE
    lane = jnp.arange(16)
    R = (jnp.arange(64)[None, :] // 8 == jnp.arange(16)[:, None]).astype(f32)
    S = ((jnp.arange(64)[:, None] % 8 == lane[None, :])
         & (lane[None, :] < 8)).astype(f32)
    c = (lane == 8).astype(f32)[None, :]
    body = functools.partial(_msg_body, n_valid=n_valid)
    return pl.pallas_call(
        body,
        grid=(grid,),
        in_specs=[
            pl.BlockSpec((BE // 32, 128), lambda i: (i, 0)),
            pl.BlockSpec((CE, 128), lambda i: (i, 0)),
            pl.BlockSpec((4, 64), lambda i: (0, 0)),
            pl.BlockSpec((1, 64), lambda i: (0, 0)),
            pl.BlockSpec((64, 64), lambda i: (0, 0)),
            pl.BlockSpec((1, 64), lambda i: (0, 0)),
            pl.BlockSpec((16, 64), lambda i: (0, 0)),
            pl.BlockSpec((64, 16), lambda i: (0, 0)),
            pl.BlockSpec((1, 16), lambda i: (0, 0)),
        ],
        out_specs=pl.BlockSpec((CE, 128), lambda i: (i, 0)),
        out_shape=jax.ShapeDtypeStruct((EPAD // 8, 128), f32),
    )(ea128, xj128, W1p, b1p, W2, b2p, R, S, c)


def _agg_body(p_ref, x_ref, root_ref, bias_ref, k_ref, m_ref, lw_ref, lb_ref,
              o_ref, *, final):
    def unpack(w):
        return jnp.concatenate([w[:, c * 16:(c + 1) * 16] for c in range(8)],
                               axis=0)

    s = unpack(p_ref[0]) + unpack(p_ref[1])
    cnt = jnp.dot(s, k_ref[...], preferred_element_type=f32)
    recip = 1.0 / jnp.maximum(cnt, 1.0)
    x16 = unpack(x_ref[...])
    h = jnp.maximum(
        s * recip * m_ref[...]
        + jnp.dot(x16, root_ref[...], preferred_element_type=f32)
        + bias_ref[...], 0.0)
    if final:
        o_ref[...] = jnp.dot(h, lw_ref[...],
                             preferred_element_type=f32) + lb_ref[...]
    else:
        o_ref[...] = jnp.concatenate(
            [h[c * CN:(c + 1) * CN, :] for c in range(8)], axis=1)


def _aggregate(p, x128, rootP, biasP, linWP, linb, final):
    lane = jnp.arange(16)
    K = (lane[:, None] == 8).astype(f32) * jnp.ones((16, 16), f32)
    M = (lane < 8).astype(f32)[None, :]
    grid = NP // 8 // CN
    body = functools.partial(_agg_body, final=final)
    if final:
        out_spec = pl.BlockSpec((CN * 8, 16), lambda i: (i, 0))
        out_shape = jax.ShapeDtypeStruct((NP, 16), f32)
    else:
        out_spec = pl.BlockSpec((CN, 128), lambda i: (i, 0))
        out_shape = jax.ShapeDtypeStruct((NP // 8, 128), f32)
    return pl.pallas_call(
        body,
        grid=(grid,),
        in_specs=[
            pl.BlockSpec((2, CN, 128), lambda i: (0, i, 0)),
            pl.BlockSpec((CN, 128), lambda i: (i, 0)),
            pl.BlockSpec((16, 16), lambda i: (0, 0)),
            pl.BlockSpec((1, 16), lambda i: (0, 0)),
            pl.BlockSpec((16, 16), lambda i: (0, 0)),
            pl.BlockSpec((1, 16), lambda i: (0, 0)),
            pl.BlockSpec((16, 16), lambda i: (0, 0)),
            pl.BlockSpec((1, 16), lambda i: (0, 0)),
        ],
        out_specs=out_spec,
        out_shape=out_shape,
    )(p, x128, rootP, biasP, K, M, linWP, linb)


def _pack_nodes(a16):
    return (a16.reshape(NP // 8 // CN, 8, CN, 16)
            .transpose(0, 2, 1, 3)
            .reshape(NP // 8, 128))


def kernel(x, edge_index, edge_attr,
           en1_W1, en1_b1, en1_W2, en1_b2, root1, bias1,
           en2_W1, en2_b1, en2_W2, en2_b2, root2, bias2,
           lin_W, lin_b):
    src = edge_index[0]
    dst = edge_index[1]

    PAD = EPAD - E
    padidx = (jnp.arange(PAD, dtype=jnp.int32) * 61) % N
    src3 = jnp.concatenate([src, padidx]).reshape(NW * STEPS, 128)
    dst3 = jnp.concatenate([dst, padidx]).reshape(NW * STEPS, 128)
    x128 = _pack_nodes(jnp.pad(x, ((0, NP - N), (0, 16 - IN_CH))))
    ea128 = (jnp.pad(edge_attr, ((0, PAD), (0, 0)))
             .reshape(EPAD // BE, 32, BE // 32, EDGE_DIM)
             .transpose(0, 2, 1, 3)
             .reshape(EPAD // 32, 128))
    W1p_1 = en1_W1
    W1p_2 = en2_W1
    root1P = jnp.pad(root1, ((0, 8), (0, 8)))
    root2P = jnp.pad(root2, ((0, 8), (0, 8)))
    bias1P = jnp.pad(bias1, (0, 8))[None, :]
    bias2P = jnp.pad(bias2, (0, 8))[None, :]
    linWP = jnp.pad(lin_W, ((0, 8), (0, 0)))
    linb = lin_b[None, :]
    zero16 = jnp.zeros((16,), f32)[None, :]

    def layer(t128, W1p, b1, W2, b2, rootP, biasP, final):
        xj128 = _sc_gather(t128, src3)
        msg = _edge_messages(ea128, xj128, W1p, b1[None, :], W2, b2[None, :], E)
        p = _sc_scatter(msg, dst3)
        return _aggregate(p, t128, rootP, biasP,
                          linWP if final else jnp.zeros((16, 16), f32),
                          linb if final else zero16, final)

    h1 = layer(x128, W1p_1, en1_b1, en1_W2, en1_b2, root1P, bias1P, False)
    out = layer(h1, W1p_2, en2_b1, en2_W2, en2_b2, root2P, bias2P, True)
    return out[:N]


# speedup vs baseline: 6.0278x; 1.0127x over previous
import functools

import jax
import jax.numpy as jnp
from jax import lax
from jax.experimental import pallas as pl
from jax.experimental.pallas import tpu as pltpu
from jax.experimental.pallas import tpu_sc as plsc

N = 50000
E = 800000
IN_CH = 8
HID_CH = 8
EDGE_DIM = 4
NUM_CLASSES = 16

NW = 32
STEPS = 196
GROUP = 7
NG = STEPS // GROUP
EPAD = NW * STEPS * 128
NP = 50176
ROWS_PER_TILE = NP // 16
TR8 = NP // 8 // 16
CE = 224
CN = TR8 // 7
BE = 1792

f32 = jnp.float32


def _sc_mesh():
    return plsc.VectorSubcoreMesh(core_axis_name="c", subcore_axis_name="s",
                                  num_cores=2, num_subcores=16)


def _sc_gather(t128, idx3, dummy):
    @functools.partial(
        pl.kernel, mesh=_sc_mesh(),
        out_type=jax.ShapeDtypeStruct((EPAD // 8, 128), f32),
        compiler_params=pltpu.CompilerParams(use_tc_tiling_on_sc=False),
        scratch_types=[
            [pltpu.VMEM((GROUP, 128), jnp.int32)] * 2,
            [pltpu.VMEM((GROUP * 128, 16), f32)] * 2,
            [pltpu.VMEM((CE, 64), f32)] * 2,
            pltpu.VMEM_SHARED((NP, 16), f32),
            [pltpu.SemaphoreType.DMA] * 2,
        ])
    def k(tbl_hbm, idx_hbm, dummy_hbm, out_hbm, idx_v, stage_v, pack_v,
          tbl_sh, sem):
        cid = lax.axis_index("c")
        sid = lax.axis_index("s")
        wid = sid * 2 + cid
        r8 = sid * TR8

        @pl.loop(0, 7)
        def _tstage(ti):
            woff = r8 + ti * CN
            pltpu.sync_copy(tbl_hbm.at[pl.ds(woff, CN), pl.ds(0, 64)],
                            pack_v[0].at[pl.ds(0, CN)])
            pltpu.sync_copy(tbl_hbm.at[pl.ds(woff, CN), pl.ds(64, 64)],
                            pack_v[1].at[pl.ds(0, CN)])

            @pl.loop(0, CN, unroll=14)
            def _trepack(q):
                for c in range(4):
                    stage_v[0][c * CN + q, :] = pack_v[0][q, pl.ds(c * 16, 16)]
                    stage_v[0][(c + 4) * CN + q, :] = (
                        pack_v[1][q, pl.ds(c * 16, 16)])

            pltpu.sync_copy(stage_v[0].at[pl.ds(0, CN * 8)],
                            tbl_sh.at[pl.ds(woff * 8, CN * 8)])

        plsc.subcore_barrier()

        def fire(g, b):
            pltpu.sync_copy(
                idx_hbm.at[pl.ds(wid * STEPS + g * GROUP, GROUP)], idx_v[b])
            for j in range(GROUP):
                pltpu.async_copy(tbl_sh.at[idx_v[b].at[j]],
                                 stage_v[b].at[pl.ds(j * 128, 128)], sem[b])

        def drain(b):
            for j in range(GROUP):
                pltpu.make_async_copy(
                    dummy_hbm, stage_v[b].at[pl.ds(j * 128, 128)],
                    sem[b]).wait()

        def repack_out(p, h):
            @pl.loop(0, CE, unroll=16)
            def _repack(q):
                for c in range(4):
                    pack_v[h][q, pl.ds(c * 16, 16)] = stage_v[h][c * CE + q, :]

            base8 = wid * (STEPS * 16) + p * CE
            pltpu.sync_copy(
                pack_v[h],
                out_hbm.at[pl.ds(base8, CE), pl.ds(h * 64, 64)])

        fire(0, 0)

        @pl.loop(0, NG // 2)
        def _pair(p):
            fire(2 * p + 1, 1)
            drain(0)
            repack_out(p, 0)

            @pl.when(p < NG // 2 - 1)
            def _():
                fire(2 * p + 2, 0)

            drain(1)
            repack_out(p, 1)

    return k(t128, idx3, dummy)


def _sc_scatter(m128, idx3, dummy):
    @functools.partial(
        pl.kernel, mesh=_sc_mesh(),
        out_type=jax.ShapeDtypeStruct((2, NP // 8, 128), f32),
        compiler_params=pltpu.CompilerParams(use_tc_tiling_on_sc=False),
        scratch_types=[
            [pltpu.VMEM((GROUP, 128), jnp.int32)] * 2,
            [pltpu.VMEM((GROUP * 128, 16), f32)] * 2,
            [pltpu.VMEM((CE, 64), f32)] * 2,
            pltpu.VMEM_SHARED((NP, 16), f32),
            [pltpu.SemaphoreType.DMA] * 2,
        ])
    def k(msg_hbm, idx_hbm, dummy_hbm, out_hbm, idx_v, stage_v, pack_v,
          acc_sh, sem):
        cid = lax.axis_index("c")
        sid = lax.axis_index("s")
        wid = sid * 2 + cid
        r8 = sid * TR8

        @pl.loop(0, GROUP * 128, unroll=16)
        def _zero(q):
            stage_v[0][q, :] = jnp.zeros((16,), f32)

        @pl.loop(0, 4)
        def _zfill(zi):
            pltpu.sync_copy(stage_v[0].at[pl.ds(0, TR8 * 2)],
                            acc_sh.at[pl.ds(r8 * 8 + zi * TR8 * 2, TR8 * 2)])

        def load_repack(g, h, b):
            pltpu.sync_copy(
                idx_hbm.at[pl.ds(wid * STEPS + g * GROUP, GROUP)], idx_v[b])
            base8 = wid * (STEPS * 16) + (g // 2) * CE
            pltpu.sync_copy(
                msg_hbm.at[pl.ds(base8, CE), pl.ds(h * 64, 64)], pack_v[b])

            @pl.loop(0, CE, unroll=16)
            def _repack(q):
                for c in range(4):
                    stage_v[b][c * CE + q, :] = pack_v[b][q, pl.ds(c * 16, 16)]

        def fire(b):
            for j in range(GROUP):
                pltpu.async_copy(stage_v[b].at[pl.ds(j * 128, 128)],
                                 acc_sh.at[idx_v[b].at[j]], sem[b], add=True)

        def drain(b):
            for j in range(GROUP):
                pltpu.make_async_copy(
                    dummy_hbm, stage_v[b].at[pl.ds(j * 128, 128)],
                    sem[b]).wait()

        plsc.subcore_barrier()
        load_repack(0, 0, 0)
        fire(0)

        @pl.loop(0, NG // 2)
        def _pair(p):
            load_repack(2 * p + 1, 1, 1)
            drain(0)
            fire(1)

            @pl.when(p < NG // 2 - 1)
            def _():
                load_repack(2 * p + 2, 0, 0)

            drain(1)

            @pl.when(p < NG // 2 - 1)
            def _():
                fire(0)

        plsc.subcore_barrier()

        @pl.loop(0, 7)
        def _out(ti):
            woff = r8 + ti * CN
            pltpu.sync_copy(acc_sh.at[pl.ds(woff * 8, CN * 8)],
                            stage_v[0].at[pl.ds(0, CN * 8)])

            @pl.loop(0, CN, unroll=14)
            def _orepack(q):
                for c in range(4):
                    pack_v[0][q, pl.ds(c * 16, 16)] = stage_v[0][c * CN + q, :]
                    pack_v[1][q, pl.ds(c * 16, 16)] = (
                        stage_v[0][(c + 4) * CN + q, :])

            pltpu.sync_copy(pack_v[0].at[pl.ds(0, CN)],
                            out_hbm.at[cid, pl.ds(woff, CN), pl.ds(0, 64)])
            pltpu.sync_copy(pack_v[1].at[pl.ds(0, CN)],
                            out_hbm.at[cid, pl.ds(woff, CN), pl.ds(64, 64)])

    return k(m128, idx3, dummy)


def _msg_body(ea_ref, xj_ref, w1_ref, b1_ref, w2_ref, b2_ref, r_ref, s_ref,
              c_ref, o_ref, *, n_valid):
    i = pl.program_id(0)
    ea128 = ea_ref[...]
    ea = jnp.concatenate([ea128[:, c * 4:(c + 1) * 4] for c in range(32)],
                         axis=0)
    xj128 = xj_ref[...]
    xj = jnp.concatenate([xj128[:, c * 16:(c + 1) * 16] for c in range(8)],
                         axis=0)
    g = jnp.maximum(
        jnp.dot(ea, w1_ref[...], preferred_element_type=f32) + b1_ref[...],
        0.0)
    h = jnp.dot(g, w2_ref[...], preferred_element_type=f32) + b2_ref[...]
    xr = jnp.dot(xj, r_ref[...], preferred_element_type=f32)
    msg = jnp.dot(h * xr, s_ref[...], preferred_element_type=f32) + c_ref[...]
    row = i * BE + jax.lax.broadcasted_iota(jnp.int32, (BE, 16), 0)
    msg = jnp.where(row < n_valid, msg, 0.0)
    o_ref[...] = jnp.concatenate(
        [msg[c * CE:(c + 1) * CE, :] for c in range(8)], axis=1)


def _edge_messages(ea128, xj128, W1p, b1p, W2, b2p, n_valid):
    grid = EPAD // BE
    lane = jnp.arange(16)
    R = (jnp.arange(64)[None, :] // 8 == jnp.arange(16)[:, None]).astype(f32)
    S = ((jnp.arange(64)[:, None] % 8 == lane[None, :])
         & (lane[None, :] < 8)).astype(f32)
    c = (lane == 8).astype(f32)[None, :]
    body = functools.partial(_msg_body, n_valid=n_valid)
    return pl.pallas_call(
        body,
        grid=(grid,),
        in_specs=[
            pl.BlockSpec((BE // 32, 128), lambda i: (i, 0)),
            pl.BlockSpec((CE, 128), lambda i: (i, 0)),
            pl.BlockSpec((4, 64), lambda i: (0, 0)),
            pl.BlockSpec((1, 64), lambda i: (0, 0)),
            pl.BlockSpec((64, 64), lambda i: (0, 0)),
            pl.BlockSpec((1, 64), lambda i: (0, 0)),
            pl.BlockSpec((16, 64), lambda i: (0, 0)),
            pl.BlockSpec((64, 16), lambda i: (0, 0)),
            pl.BlockSpec((1, 16), lambda i: (0, 0)),
        ],
        out_specs=pl.BlockSpec((CE, 128), lambda i: (i, 0)),
        out_shape=jax.ShapeDtypeStruct((EPAD // 8, 128), f32),
    )(ea128, xj128, W1p, b1p, W2, b2p, R, S, c)


def _agg_body(p_ref, x_ref, root_ref, bias_ref, k_ref, m_ref, lw_ref, lb_ref,
              o_ref, *, final):
    def unpack(w):
        return jnp.concatenate([w[:, c * 16:(c + 1) * 16] for c in range(8)],
                               axis=0)

    s = unpack(p_ref[0]) + unpack(p_ref[1])
    cnt = jnp.dot(s, k_ref[...], preferred_element_type=f32)
    recip = 1.0 / jnp.maximum(cnt, 1.0)
    x16 = unpack(x_ref[...])
    h = jnp.maximum(
        s * recip * m_ref[...]
        + jnp.dot(x16, root_ref[...], preferred_element_type=f32)
        + bias_ref[...], 0.0)
    if final:
        o_ref[...] = jnp.dot(h, lw_ref[...],
                             preferred_element_type=f32) + lb_ref[...]
    else:
        o_ref[...] = jnp.concatenate(
            [h[c * CN:(c + 1) * CN, :] for c in range(8)], axis=1)


def _aggregate(p, x128, rootP, biasP, linWP, linb, final):
    lane = jnp.arange(16)
    K = (lane[:, None] == 8).astype(f32) * jnp.ones((16, 16), f32)
    M = (lane < 8).astype(f32)[None, :]
    grid = NP // 8 // CN
    body = functools.partial(_agg_body, final=final)
    if final:
        out_spec = pl.BlockSpec((CN * 8, 16), lambda i: (i, 0))
        out_shape = jax.ShapeDtypeStruct((NP, 16), f32)
    else:
        out_spec = pl.BlockSpec((CN, 128), lambda i: (i, 0))
        out_shape = jax.ShapeDtypeStruct((NP // 8, 128), f32)
    return pl.pallas_call(
        body,
        grid=(grid,),
        in_specs=[
            pl.BlockSpec((2, CN, 128), lambda i: (0, i, 0)),
            pl.BlockSpec((CN, 128), lambda i: (i, 0)),
            pl.BlockSpec((16, 16), lambda i: (0, 0)),
            pl.BlockSpec((1, 16), lambda i: (0, 0)),
            pl.BlockSpec((16, 16), lambda i: (0, 0)),
            pl.BlockSpec((1, 16), lambda i: (0, 0)),
            pl.BlockSpec((16, 16), lambda i: (0, 0)),
            pl.BlockSpec((1, 16), lambda i: (0, 0)),
        ],
        out_specs=out_spec,
        out_shape=out_shape,
    )(p, x128, rootP, biasP, K, M, linWP, linb)


def _pack_nodes(a16):
    return (a16.reshape(NP // 8 // CN, 8, CN, 16)
            .transpose(0, 2, 1, 3)
            .reshape(NP // 8, 128))


def kernel(x, edge_index, edge_attr,
           en1_W1, en1_b1, en1_W2, en1_b2, root1, bias1,
           en2_W1, en2_b1, en2_W2, en2_b2, root2, bias2,
           lin_W, lin_b):
    src = edge_index[0]
    dst = edge_index[1]

    PAD = EPAD - E
    padidx = (jnp.arange(PAD, dtype=jnp.int32) * 61) % N
    src3 = jnp.concatenate([src, padidx]).reshape(NW * STEPS, 128)
    dst3 = jnp.concatenate([dst, padidx]).reshape(NW * STEPS, 128)
    dummy = jnp.zeros((128, 16), f32)
    x128 = _pack_nodes(jnp.pad(x, ((0, NP - N), (0, 16 - IN_CH))))
    ea128 = (jnp.pad(edge_attr, ((0, PAD), (0, 0)))
             .reshape(EPAD // BE, 32, BE // 32, EDGE_DIM)
             .transpose(0, 2, 1, 3)
             .reshape(EPAD // 32, 128))
    W1p_1 = en1_W1
    W1p_2 = en2_W1
    root1P = jnp.pad(root1, ((0, 8), (0, 8)))
    root2P = jnp.pad(root2, ((0, 8), (0, 8)))
    bias1P = jnp.pad(bias1, (0, 8))[None, :]
    bias2P = jnp.pad(bias2, (0, 8))[None, :]
    linWP = jnp.pad(lin_W, ((0, 8), (0, 0)))
    linb = lin_b[None, :]
    zero16 = jnp.zeros((16,), f32)[None, :]

    def layer(t128, W1p, b1, W2, b2, rootP, biasP, final):
        xj128 = _sc_gather(t128, src3, dummy)
        msg = _edge_messages(ea128, xj128, W1p, b1[None, :], W2, b2[None, :], E)
        p = _sc_scatter(msg, dst3, dummy)
        return _aggregate(p, t128, rootP, biasP,
                          linWP if final else jnp.zeros((16, 16), f32),
                          linb if final else zero16, final)

    h1 = layer(x128, W1p_1, en1_b1, en1_W2, en1_b2, root1P, bias1P, False)
    out = layer(h1, W1p_2, en2_b1, en2_W2, en2_b2, root2P, bias2P, True)
    return out[:N]


# speedup vs baseline: 6.0457x; 1.0030x over previous
import functools

import jax
import jax.numpy as jnp
from jax import lax
from jax.experimental import pallas as pl
from jax.experimental.pallas import tpu as pltpu
from jax.experimental.pallas import tpu_sc as plsc

N = 50000
E = 800000
IN_CH = 8
HID_CH = 8
EDGE_DIM = 4
NUM_CLASSES = 16

NW = 32
STEPS = 196
GROUP = 7
NG = STEPS // GROUP
EPAD = NW * STEPS * 128
NP = 50176
ROWS_PER_TILE = NP // 16
TR8 = NP // 8 // 16
CE = 224
CN = TR8 // 7
BE = 1792

f32 = jnp.float32


def _sc_mesh():
    return plsc.VectorSubcoreMesh(core_axis_name="c", subcore_axis_name="s",
                                  num_cores=2, num_subcores=16)


def _sc_gather(t128, idx3, dummy):
    @functools.partial(
        pl.kernel, mesh=_sc_mesh(),
        out_type=jax.ShapeDtypeStruct((EPAD // 8, 128), f32),
        compiler_params=pltpu.CompilerParams(use_tc_tiling_on_sc=False),
        scratch_types=[
            [pltpu.VMEM((GROUP, 128), jnp.int32)] * 2,
            [pltpu.VMEM((GROUP * 128, 16), f32)] * 2,
            [pltpu.VMEM((CE, 64), f32)] * 2,
            pltpu.VMEM_SHARED((NP, 16), f32),
            [pltpu.SemaphoreType.DMA] * 2,
        ])
    def k(tbl_hbm, idx_hbm, dummy_hbm, out_hbm, idx_v, stage_v, pack_v,
          tbl_sh, sem):
        cid = lax.axis_index("c")
        sid = lax.axis_index("s")
        wid = sid * 2 + cid
        r8 = sid * TR8

        @pl.loop(0, 7)
        def _tstage(ti):
            woff = r8 + ti * CN
            pltpu.sync_copy(tbl_hbm.at[pl.ds(woff, CN), pl.ds(0, 64)],
                            pack_v[0].at[pl.ds(0, CN)])
            pltpu.sync_copy(tbl_hbm.at[pl.ds(woff, CN), pl.ds(64, 64)],
                            pack_v[1].at[pl.ds(0, CN)])

            @pl.loop(0, CN, unroll=14)
            def _trepack(q):
                for c in range(4):
                    stage_v[0][c * CN + q, :] = pack_v[0][q, pl.ds(c * 16, 16)]
                    stage_v[0][(c + 4) * CN + q, :] = (
                        pack_v[1][q, pl.ds(c * 16, 16)])

            pltpu.sync_copy(stage_v[0].at[pl.ds(0, CN * 8)],
                            tbl_sh.at[pl.ds(woff * 8, CN * 8)])

        plsc.subcore_barrier()

        def fire(g, b):
            pltpu.sync_copy(
                idx_hbm.at[pl.ds(wid * STEPS + g * GROUP, GROUP)], idx_v[b])
            for j in range(GROUP):
                pltpu.async_copy(tbl_sh.at[idx_v[b].at[j]],
                                 stage_v[b].at[pl.ds(j * 128, 128)], sem[b])

        def drain(b):
            pltpu.make_async_copy(dummy_hbm, stage_v[b], sem[b]).wait()

        def repack_out(p, h):
            @pl.loop(0, CE, unroll=16)
            def _repack(q):
                for c in range(4):
                    pack_v[h][q, pl.ds(c * 16, 16)] = stage_v[h][c * CE + q, :]

            base8 = wid * (STEPS * 16) + p * CE
            pltpu.sync_copy(
                pack_v[h],
                out_hbm.at[pl.ds(base8, CE), pl.ds(h * 64, 64)])

        fire(0, 0)

        @pl.loop(0, NG // 2)
        def _pair(p):
            fire(2 * p + 1, 1)
            drain(0)
            repack_out(p, 0)

            @pl.when(p < NG // 2 - 1)
            def _():
                fire(2 * p + 2, 0)

            drain(1)
            repack_out(p, 1)

    return k(t128, idx3, dummy)


def _sc_scatter(m128, idx3, dummy):
    @functools.partial(
        pl.kernel, mesh=_sc_mesh(),
        out_type=jax.ShapeDtypeStruct((2, NP // 8, 128), f32),
        compiler_params=pltpu.CompilerParams(use_tc_tiling_on_sc=False),
        scratch_types=[
            [pltpu.VMEM((GROUP, 128), jnp.int32)] * 2,
            [pltpu.VMEM((GROUP * 128, 16), f32)] * 2,
            [pltpu.VMEM((CE, 64), f32)] * 2,
            pltpu.VMEM_SHARED((NP, 16), f32),
            [pltpu.SemaphoreType.DMA] * 2,
        ])
    def k(msg_hbm, idx_hbm, dummy_hbm, out_hbm, idx_v, stage_v, pack_v,
          acc_sh, sem):
        cid = lax.axis_index("c")
        sid = lax.axis_index("s")
        wid = sid * 2 + cid
        r8 = sid * TR8

        @pl.loop(0, GROUP * 128, unroll=16)
        def _zero(q):
            stage_v[0][q, :] = jnp.zeros((16,), f32)

        @pl.loop(0, 4)
        def _zfill(zi):
            pltpu.sync_copy(stage_v[0].at[pl.ds(0, TR8 * 2)],
                            acc_sh.at[pl.ds(r8 * 8 + zi * TR8 * 2, TR8 * 2)])

        def load_repack(g, h, b):
            pltpu.sync_copy(
                idx_hbm.at[pl.ds(wid * STEPS + g * GROUP, GROUP)], idx_v[b])
            base8 = wid * (STEPS * 16) + (g // 2) * CE
            pltpu.sync_copy(
                msg_hbm.at[pl.ds(base8, CE), pl.ds(h * 64, 64)], pack_v[b])

            @pl.loop(0, CE, unroll=16)
            def _repack(q):
                for c in range(4):
                    stage_v[b][c * CE + q, :] = pack_v[b][q, pl.ds(c * 16, 16)]

        def fire(b):
            for j in range(GROUP):
                pltpu.async_copy(stage_v[b].at[pl.ds(j * 128, 128)],
                                 acc_sh.at[idx_v[b].at[j]], sem[b], add=True)

        def drain(b):
            pltpu.make_async_copy(dummy_hbm, stage_v[b], sem[b]).wait()

        plsc.subcore_barrier()
        load_repack(0, 0, 0)
        fire(0)

        @pl.loop(0, NG // 2)
        def _pair(p):
            load_repack(2 * p + 1, 1, 1)
            drain(0)
            fire(1)

            @pl.when(p < NG // 2 - 1)
            def _():
                load_repack(2 * p + 2, 0, 0)

            drain(1)

            @pl.when(p < NG // 2 - 1)
            def _():
                fire(0)

        plsc.subcore_barrier()

        @pl.loop(0, 7)
        def _out(ti):
            woff = r8 + ti * CN
            pltpu.sync_copy(acc_sh.at[pl.ds(woff * 8, CN * 8)],
                            stage_v[0].at[pl.ds(0, CN * 8)])

            @pl.loop(0, CN, unroll=14)
            def _orepack(q):
                for c in range(4):
                    pack_v[0][q, pl.ds(c * 16, 16)] = stage_v[0][c * CN + q, :]
                    pack_v[1][q, pl.ds(c * 16, 16)] = (
                        stage_v[0][(c + 4) * CN + q, :])

            pltpu.sync_copy(pack_v[0].at[pl.ds(0, CN)],
                            out_hbm.at[cid, pl.ds(woff, CN), pl.ds(0, 64)])
            pltpu.sync_copy(pack_v[1].at[pl.ds(0, CN)],
                            out_hbm.at[cid, pl.ds(woff, CN), pl.ds(64, 64)])

    return k(m128, idx3, dummy)


def _msg_body(ea_ref, xj_ref, w1_ref, b1_ref, w2_ref, b2_ref, r_ref, s_ref,
              c_ref, o_ref, *, n_valid):
    i = pl.program_id(0)
    ea128 = ea_ref[...]
    ea = jnp.concatenate([ea128[:, c * 4:(c + 1) * 4] for c in range(32)],
                         axis=0)
    xj128 = xj_ref[...]
    xj = jnp.concatenate([xj128[:, c * 16:(c + 1) * 16] for c in range(8)],
                         axis=0)
    g = jnp.maximum(
        jnp.dot(ea, w1_ref[...], preferred_element_type=f32) + b1_ref[...],
        0.0)
    h = jnp.dot(g, w2_ref[...], preferred_element_type=f32) + b2_ref[...]
    xr = jnp.dot(xj, r_ref[...], preferred_element_type=f32)
    msg = jnp.dot(h * xr, s_ref[...], preferred_element_type=f32) + c_ref[...]
    row = i * BE + jax.lax.broadcasted_iota(jnp.int32, (BE, 16), 0)
    msg = jnp.where(row < n_valid, msg, 0.0)
    o_ref[...] = jnp.concatenate(
        [msg[c * CE:(c + 1) * CE, :] for c in range(8)], axis=1)


def _edge_messages(ea128, xj128, W1p, b1p, W2, b2p, n_valid):
    grid = EPAD // BE
    lane = jnp.arange(16)
    R = (jnp.arange(64)[None, :] // 8 == jnp.arange(16)[:, None]).astype(f32)
    S = ((jnp.arange(64)[:, None] % 8 == lane[None, :])
         & (lane[None, :] < 8)).astype(f32)
    c = (lane == 8).astype(f32)[None, :]
    body = functools.partial(_msg_body, n_valid=n_valid)
    return pl.pallas_call(
        body,
        grid=(grid,),
        in_specs=[
            pl.BlockSpec((BE // 32, 128), lambda i: (i, 0)),
            pl.BlockSpec((CE, 128), lambda i: (i, 0)),
            pl.BlockSpec((4, 64), lambda i: (0, 0)),
            pl.BlockSpec((1, 64), lambda i: (0, 0)),
            pl.BlockSpec((64, 64), lambda i: (0, 0)),
            pl.BlockSpec((1, 64), lambda i: (0, 0)),
            pl.BlockSpec((16, 64), lambda i: (0, 0)),
            pl.BlockSpec((64, 16), lambda i: (0, 0)),
            pl.BlockSpec((1, 16), lambda i: (0, 0)),
        ],
        out_specs=pl.BlockSpec((CE, 128), lambda i: (i, 0)),
        out_shape=jax.ShapeDtypeStruct((EPAD // 8, 128), f32),
    )(ea128, xj128, W1p, b1p, W2, b2p, R, S, c)


def _agg_body(p_ref, x_ref, root_ref, bias_ref, k_ref, m_ref, lw_ref, lb_ref,
              o_ref, *, final):
    def unpack(w):
        return jnp.concatenate([w[:, c * 16:(c + 1) * 16] for c in range(8)],
                               axis=0)

    s = unpack(p_ref[0]) + unpack(p_ref[1])
    cnt = jnp.dot(s, k_ref[...], preferred_element_type=f32)
    recip = 1.0 / jnp.maximum(cnt, 1.0)
    x16 = unpack(x_ref[...])
    h = jnp.maximum(
        s * recip * m_ref[...]
        + jnp.dot(x16, root_ref[...], preferred_element_type=f32)
        + bias_ref[...], 0.0)
    if final:
        o_ref[...] = jnp.dot(h, lw_ref[...],
                             preferred_element_type=f32) + lb_ref[...]
    else:
        o_ref[...] = jnp.concatenate(
            [h[c * CN:(c + 1) * CN, :] for c in range(8)], axis=1)


def _aggregate(p, x128, rootP, biasP, linWP, linb, final):
    lane = jnp.arange(16)
    K = (lane[:, None] == 8).astype(f32) * jnp.ones((16, 16), f32)
    M = (lane < 8).astype(f32)[None, :]
    grid = NP // 8 // CN
    body = functools.partial(_agg_body, final=final)
    if final:
        out_spec = pl.BlockSpec((CN * 8, 16), lambda i: (i, 0))
        out_shape = jax.ShapeDtypeStruct((NP, 16), f32)
    else:
        out_spec = pl.BlockSpec((CN, 128), lambda i: (i, 0))
        out_shape = jax.ShapeDtypeStruct((NP // 8, 128), f32)
    return pl.pallas_call(
        body,
        grid=(grid,),
        in_specs=[
            pl.BlockSpec((2, CN, 128), lambda i: (0, i, 0)),
            pl.BlockSpec((CN, 128), lambda i: (i, 0)),
            pl.BlockSpec((16, 16), lambda i: (0, 0)),
            pl.BlockSpec((1, 16), lambda i: (0, 0)),
            pl.BlockSpec((16, 16), lambda i: (0, 0)),
            pl.BlockSpec((1, 16), lambda i: (0, 0)),
            pl.BlockSpec((16, 16), lambda i: (0, 0)),
            pl.BlockSpec((1, 16), lambda i: (0, 0)),
        ],
        out_specs=out_spec,
        out_shape=out_shape,
    )(p, x128, rootP, biasP, K, M, linWP, linb)


def _pack_nodes(a16):
    return (a16.reshape(NP // 8 // CN, 8, CN, 16)
            .transpose(0, 2, 1, 3)
            .reshape(NP // 8, 128))


def kernel(x, edge_index, edge_attr,
           en1_W1, en1_b1, en1_W2, en1_b2, root1, bias1,
           en2_W1, en2_b1, en2_W2, en2_b2, root2, bias2,
           lin_W, lin_b):
    src = edge_index[0]
    dst = edge_index[1]

    PAD = EPAD - E
    padidx = (jnp.arange(PAD, dtype=jnp.int32) * 61) % N
    src3 = jnp.concatenate([src, padidx]).reshape(NW * STEPS, 128)
    dst3 = jnp.concatenate([dst, padidx]).reshape(NW * STEPS, 128)
    dummy = jnp.zeros((GROUP * 128, 16), f32)
    x128 = _pack_nodes(jnp.pad(x, ((0, NP - N), (0, 16 - IN_CH))))
    ea128 = (jnp.pad(edge_attr, ((0, PAD), (0, 0)))
             .reshape(EPAD // BE, 32, BE // 32, EDGE_DIM)
             .transpose(0, 2, 1, 3)
             .reshape(EPAD // 32, 128))
    W1p_1 = en1_W1
    W1p_2 = en2_W1
    root1P = jnp.pad(root1, ((0, 8), (0, 8)))
    root2P = jnp.pad(root2, ((0, 8), (0, 8)))
    bias1P = jnp.pad(bias1, (0, 8))[None, :]
    bias2P = jnp.pad(bias2, (0, 8))[None, :]
    linWP = jnp.pad(lin_W, ((0, 8), (0, 0)))
    linb = lin_b[None, :]
    zero16 = jnp.zeros((16,), f32)[None, :]

    def layer(t128, W1p, b1, W2, b2, rootP, biasP, final):
        xj128 = _sc_gather(t128, src3, dummy)
        msg = _edge_messages(ea128, xj128, W1p, b1[None, :], W2, b2[None, :], E)
        p = _sc_scatter(msg, dst3, dummy)
        return _aggregate(p, t128, rootP, biasP,
                          linWP if final else jnp.zeros((16, 16), f32),
                          linb if final else zero16, final)

    h1 = layer(x128, W1p_1, en1_b1, en1_W2, en1_b2, root1P, bias1P, False)
    out = layer(h1, W1p_2, en2_b1, en2_W2, en2_b2, root2P, bias2P, True)
    return out[:N]
